# Initial kernel scaffold; baseline (speedup 1.0000x reference)
import jax
import jax.numpy as jnp
from jax.experimental import pallas as pl


def kernel(input, target, cluster_weight, cluster_bias, proj0, proj1, proj2, w0, b0, w1, b1, w2, b2):
    raise NotImplementedError("write your pallas kernel here")


# speedup vs baseline: 2.4421x; 2.4421x over previous
import functools

import jax
import jax.numpy as jnp
from jax.experimental import pallas as pl
from jax.experimental.pallas import tpu as pltpu

_NEG = -1e30


def _proj_body(x_ref, p0_ref, p1_ref, p2_ref, o0_ref, o1_ref, o2_ref):
    x = x_ref[...]
    for p_ref, o_ref in ((p0_ref, o0_ref), (p1_ref, o1_ref), (p2_ref, o2_ref)):
        o_ref[...] = jax.lax.dot_general(
            x, p_ref[...], (((1,), (0,)), ((), ())),
            preferred_element_type=jnp.float32).astype(jnp.bfloat16)


def _project(x, p0, p1, p2, *, blk_r, interpret=False):
    n, d = x.shape
    k1 = p1.shape[1]
    k2 = p2.shape[1]
    return pl.pallas_call(
        _proj_body,
        grid=(n // blk_r,),
        in_specs=[
            pl.BlockSpec((blk_r, d), lambda r: (r, 0)),
            pl.BlockSpec((d, d), lambda r: (0, 0)),
            pl.BlockSpec((d, k1), lambda r: (0, 0)),
            pl.BlockSpec((d, k2), lambda r: (0, 0)),
        ],
        out_specs=[
            pl.BlockSpec((blk_r, d), lambda r: (r, 0)),
            pl.BlockSpec((blk_r, k1), lambda r: (r, 0)),
            pl.BlockSpec((blk_r, k2), lambda r: (r, 0)),
        ],
        out_shape=[
            jax.ShapeDtypeStruct((n, d), jnp.bfloat16),
            jax.ShapeDtypeStruct((n, k1), jnp.bfloat16),
            jax.ShapeDtypeStruct((n, k2), jnp.bfloat16),
        ],
        compiler_params=pltpu.CompilerParams(
            dimension_semantics=("arbitrary",)),
        interpret=interpret,
    )(x, p0, p1, p2)


def _adaptive_nll(input, target, cluster_weight, cluster_bias,
                  proj0, proj1, proj2, w0, b0, w1, b1, w2, b2,
                  *, cut0, cut1, vocab, blk_r, blk_c, sub, interpret=False):
    n, d = input.shape
    x = input.astype(jnp.bfloat16)
    tgt = target.astype(jnp.int32).reshape(n, 1)
    ph0, ph1, ph2 = _project(x, proj0.astype(jnp.bfloat16),
                             proj1.astype(jnp.bfloat16),
                             proj2.astype(jnp.bfloat16),
                             blk_r=min(blk_r, 1024), interpret=interpret)

    hid_head = jnp.where(tgt < cut0, tgt, -1)

    def head_extras(ph_ref, s_ref, t_ref, out_ref, *, cw_ref, cb_ref,
                    tgt_ref):
        cl = jax.lax.dot_general(
            ph_ref[...], cw_ref[...], (((1,), (1,)), ((), ())),
            preferred_element_type=jnp.float32) + cb_ref[...]
        cl0 = cl[:, 0:1]
        cl1 = cl[:, 1:2]
        s = s_ref[...] + jnp.exp(cl0) + jnp.exp(cl1)
        tg = tgt_ref[...]
        t = t_ref[...] + jnp.where(tg >= cut1, cl0, 0.0) \
            + jnp.where((tg >= cut0) & (tg < cut1), cl1, 0.0)
        out_ref[...] = jnp.log(s) - t

    cwb = cluster_weight.astype(jnp.bfloat16)
    cbb = cluster_bias.reshape(1, 2)

    head_nll = _sweep_with_extras(
        ph0, w0.astype(jnp.bfloat16), b0.reshape(1, -1), hid_head,
        [cwb, cbb, tgt], head_extras,
        blk_r=blk_r, blk_c=blk_c, sub=sub, interpret=interpret)

    def tail_extras_factory(lo, hi, prev_used):
        def tail_extras(ph_ref, s_ref, t_ref, out_ref, *, tgt_ref, prev_ref):
            tg = tgt_ref[...]
            in_l = (tg >= lo) & (tg < hi)
            cnll = jnp.where(in_l, jnp.log(s_ref[...]) - t_ref[...], 0.0)
            out_ref[...] = prev_ref[...] + cnll
        return tail_extras

    hid1 = jnp.where((tgt >= cut0) & (tgt < cut1), tgt - cut0, -1)
    nll1 = _sweep_with_extras(
        ph1, w1.astype(jnp.bfloat16), b1.reshape(1, -1), hid1,
        [tgt, head_nll], tail_extras_factory(cut0, cut1, True),
        tail=True, blk_r=blk_r, blk_c=blk_c, sub=sub, interpret=interpret)
    hid2 = jnp.where(tgt >= cut1, tgt - cut1, -1)
    nll = _sweep_with_extras(
        ph2, w2.astype(jnp.bfloat16), b2.reshape(1, -1), hid2,
        [tgt, nll1], tail_extras_factory(cut1, vocab, True),
        tail=True, blk_r=blk_r, blk_c=blk_c, sub=sub, interpret=interpret)
    return nll.reshape(n)


def _sweep_with_extras(ph, w, b, hid, extra_inputs, extras_fn, *, tail=False,
                       blk_r, blk_c, sub, interpret=False):
    n, k = ph.shape
    n_cols = w.shape[0]
    n_rblk = n // blk_r
    n_cblk = pl.cdiv(n_cols, blk_c)

    def body(ph_ref, w_ref, b_ref, hid_ref, *rest):
        extra_refs = rest[:len(extra_inputs)]
        out_ref = rest[len(extra_inputs)]
        s_ref, t_ref = rest[len(extra_inputs) + 1:]

        c = pl.program_id(1)
        n_sub = blk_r // sub

        @pl.when(c == 0)
        def _init():
            s_ref[...] = jnp.zeros_like(s_ref)
            t_ref[...] = jnp.zeros_like(t_ref)

        def accum(i, masked):
            rs = pl.ds(i * sub, sub)
            logits = jax.lax.dot_general(
                ph_ref[rs, :], w_ref[...], (((1,), (1,)), ((), ())),
                preferred_element_type=jnp.float32) + b_ref[...]
            col = c * blk_c + jax.lax.broadcasted_iota(
                jnp.int32, (sub, blk_c), 1)
            if masked:
                e = jnp.exp(jnp.where(col < n_cols, logits, _NEG))
            else:
                e = jnp.exp(logits)
            s_ref[rs, :] += jnp.sum(e, axis=1, keepdims=True)
            onehot = col == hid_ref[rs, :]
            t_ref[rs, :] += jnp.sum(jnp.where(onehot, logits, 0.0),
                                    axis=1, keepdims=True)

        @pl.when(c < n_cblk - 1)
        def _full():
            jax.lax.fori_loop(0, n_sub,
                              lambda i, _: (accum(i, False), 0)[1], 0)

        @pl.when(c == n_cblk - 1)
        def _last():
            jax.lax.fori_loop(0, n_sub,
                              lambda i, _: (accum(i, True), 0)[1], 0)
            if tail:
                extras_fn(ph_ref, s_ref, t_ref, out_ref,
                          tgt_ref=extra_refs[0], prev_ref=extra_refs[1])
            else:
                extras_fn(ph_ref, s_ref, t_ref, out_ref,
                          cw_ref=extra_refs[0], cb_ref=extra_refs[1],
                          tgt_ref=extra_refs[2])

    def small_spec(a):
        if a.ndim == 2 and a.shape[0] <= 8:
            return pl.BlockSpec(a.shape, lambda r, c: (0, 0))
        return pl.BlockSpec((blk_r, a.shape[1]), lambda r, c: (r, 0))

    return pl.pallas_call(
        body,
        grid=(n_rblk, n_cblk),
        in_specs=[
            pl.BlockSpec((blk_r, k), lambda r, c: (r, 0)),
            pl.BlockSpec((blk_c, k), lambda r, c: (c, 0)),
            pl.BlockSpec((1, blk_c), lambda r, c: (0, c)),
            pl.BlockSpec((blk_r, 1), lambda r, c: (r, 0)),
        ] + [small_spec(a) for a in extra_inputs],
        out_specs=pl.BlockSpec((blk_r, 1), lambda r, c: (r, 0)),
        out_shape=jax.ShapeDtypeStruct((n, 1), jnp.float32),
        scratch_shapes=[
            pltpu.VMEM((blk_r, 1), jnp.float32),
            pltpu.VMEM((blk_r, 1), jnp.float32),
        ],
        compiler_params=pltpu.CompilerParams(
            dimension_semantics=("arbitrary", "arbitrary")),
        interpret=interpret,
    )(ph, w, b, hid, *extra_inputs)


def kernel(input, target, cluster_weight, cluster_bias, proj0, proj1, proj2,
           w0, b0, w1, b1, w2, b2):
    return _adaptive_nll(
        input, target, cluster_weight, cluster_bias,
        proj0, proj1, proj2, w0, b0, w1, b1, w2, b2,
        cut0=20000, cut1=60000, vocab=100000,
        blk_r=4096, blk_c=1024, sub=512)
